# baseline (speedup 1.0000x reference)
import functools

import jax
import jax.numpy as jnp
from jax import lax
from jax.experimental import pallas as pl
from jax.experimental.pallas import tpu as pltpu
from jax.experimental.pallas import tpu_sc as plsc

T = 2048
D = 1024
E = 8
F = 4096
EPAD = 128
TILE = 256
NT = 15
NTPAD = NT * TILE
NW = 32
BPW = NTPAD // NW
TRASH = T
NJ = 4


def _router_body(x_ref, wr_ref, logits_ref, eidx_ref, prob_ref):
    x = x_ref[...].astype(jnp.bfloat16)
    wr = wr_ref[...].astype(jnp.bfloat16)
    logits = jnp.dot(x, wr, preferred_element_type=jnp.float32)
    lane = lax.broadcasted_iota(jnp.int32, (T, EPAD), 1)
    masked = jnp.where(lane < E, logits, -jnp.inf)
    m = jnp.max(masked, axis=-1, keepdims=True)
    ex = jnp.exp(masked - m)
    s = jnp.sum(ex, axis=-1, keepdims=True)
    probs = ex / s
    pmax = jnp.max(probs, axis=-1, keepdims=True)
    eidx = jnp.min(jnp.where(probs == pmax, lane, EPAD), axis=-1, keepdims=True)
    logits_ref[...] = logits
    eidx_ref[...] = jnp.broadcast_to(eidx, (T, EPAD))
    prob_ref[...] = jnp.broadcast_to(pmax, (T, EPAD))


def _router(x2d, wr_pad):
    return pl.pallas_call(
        _router_body,
        out_shape=(
            jax.ShapeDtypeStruct((T, EPAD), jnp.float32),
            jax.ShapeDtypeStruct((T, EPAD), jnp.int32),
            jax.ShapeDtypeStruct((T, EPAD), jnp.float32),
        ),
    )(x2d, wr_pad)


TPW = T // NW


def _sc_dispatch_body(x_hbm, idx_hbm, out_hbm, idx_v, rows_v, sem):
    wid = lax.axis_index("s") * 2 + lax.axis_index("c")
    base = wid * TPW
    pltpu.sync_copy(idx_hbm.at[pl.ds(base, TPW)], idx_v)
    pltpu.sync_copy(x_hbm.at[pl.ds(base, TPW)], rows_v)
    pltpu.async_copy(rows_v, out_hbm.at[idx_v], sem).wait()


@functools.cache
def _dispatch_kernel():
    return pl.kernel(
        _sc_dispatch_body,
        out_type=jax.ShapeDtypeStruct((NTPAD, D), jnp.float32),
        mesh=plsc.VectorSubcoreMesh(core_axis_name="c", subcore_axis_name="s"),
        scratch_types=[
            pltpu.VMEM((TPW,), jnp.int32),
            pltpu.VMEM((TPW, D), jnp.float32),
            pltpu.SemaphoreType.DMA,
        ],
    )


def _dispatch(x2d, dest_slot):
    return _dispatch_kernel()(x2d, dest_slot)


def _sc_scatter_body(ypad_hbm, idx_hbm, out_hbm, idx_v, rows_v, sem):
    wid = lax.axis_index("s") * 2 + lax.axis_index("c")
    base = wid * BPW
    pltpu.sync_copy(idx_hbm.at[pl.ds(base, BPW)], idx_v)
    pltpu.sync_copy(ypad_hbm.at[pl.ds(base, BPW)], rows_v)
    pltpu.async_copy(rows_v, out_hbm.at[idx_v], sem).wait()


@functools.cache
def _combine_kernel():
    return pl.kernel(
        _sc_scatter_body,
        out_type=jax.ShapeDtypeStruct((T + NW, D), jnp.float32),
        mesh=plsc.VectorSubcoreMesh(core_axis_name="c", subcore_axis_name="s"),
        scratch_types=[
            pltpu.VMEM((BPW,), jnp.int32),
            pltpu.VMEM((BPW, D), jnp.float32),
            pltpu.SemaphoreType.DMA,
        ],
    )


def _combine(y_pad, dest):
    return _combine_kernel()(y_pad, dest)


def _ffn_body(te_ref, na_ref, x_ref, w1_ref, b1_ref, w2_ref, b2_ref, p_ref,
              y_ref, acc_ref):
    j = pl.program_id(0)
    t = pl.program_id(1)

    @pl.when(t < na_ref[0])
    def _():
        xb = x_ref[...].astype(jnp.bfloat16)
        w1 = w1_ref[0].astype(jnp.bfloat16)
        h = jnp.dot(xb, w1, preferred_element_type=jnp.float32) + b1_ref[0]
        h = jnp.maximum(h, 0.0)
        w2 = w2_ref[0].astype(jnp.bfloat16)
        o = jnp.dot(h.astype(jnp.bfloat16), w2, preferred_element_type=jnp.float32)
        sl = pl.ds(t * TILE, TILE)

        @pl.when(j == 0)
        def _():
            acc_ref[sl, :] = o

        @pl.when(jnp.logical_and(j > 0, j < NJ - 1))
        def _():
            acc_ref[sl, :] += o

        @pl.when(j == NJ - 1)
        def _():
            y_ref[...] = (acc_ref[sl, :] + o + b2_ref[0]) * p_ref[...]


def _ffn(x_pad, w1r, b1r, w2r, b2r, prob_pad, tile_expert, nactive):
    fh = F // NJ
    grid_spec = pltpu.PrefetchScalarGridSpec(
        num_scalar_prefetch=2,
        grid=(NJ, NT),
        in_specs=[
            pl.BlockSpec((TILE, D), lambda j, t, te, na: (t, 0)),
            pl.BlockSpec((1, D, fh), lambda j, t, te, na: (te[t], 0, j)),
            pl.BlockSpec((1, 1, fh), lambda j, t, te, na: (te[t], 0, j)),
            pl.BlockSpec((1, fh, D), lambda j, t, te, na: (te[t], j, 0)),
            pl.BlockSpec((1, 1, D), lambda j, t, te, na: (te[t], 0, 0)),
            pl.BlockSpec((TILE, 1), lambda j, t, te, na: (t, 0)),
        ],
        out_specs=pl.BlockSpec(
            (TILE, D), lambda j, t, te, na: (jnp.where(j < NJ - 1, NT, t), 0)),
        scratch_shapes=[pltpu.VMEM((NTPAD, D), jnp.float32)],
    )
    out = pl.pallas_call(
        _ffn_body,
        grid_spec=grid_spec,
        out_shape=jax.ShapeDtypeStruct((NTPAD + TILE, D), jnp.float32),
    )(tile_expert, nactive, x_pad, w1r, b1r, w2r, b2r, prob_pad)
    return out[:NTPAD]


def kernel(hidden_states, Wr, W1, b1, W2, b2):
    b, s, d = hidden_states.shape
    x2d = hidden_states.reshape(T, D)

    wr_pad = jnp.zeros((D, EPAD), jnp.float32).at[:, :E].set(Wr)
    logits_pad, eidx_b, prob_b = _router(x2d, wr_pad)
    eidx = eidx_b[:, 0]
    prob_vec = prob_b[:, 0]
    router_logits = logits_pad[:, :E].reshape(b, s, E)

    onehot = (eidx[:, None] == jnp.arange(E)[None, :]).astype(jnp.int32)
    csum = jnp.cumsum(onehot, axis=0)
    rank = jnp.take_along_axis(csum, eidx[:, None], axis=1)[:, 0] - 1
    g = csum[-1]
    padded_g = ((g + (TILE - 1)) // TILE) * TILE
    pstart = jnp.concatenate(
        [jnp.zeros((1,), jnp.int32), jnp.cumsum(padded_g)[:-1]])
    dest_slot = pstart[eidx] + rank
    rows = jnp.full((NTPAD,), T, jnp.int32).at[dest_slot].set(
        jnp.arange(T, dtype=jnp.int32))
    nactive = ((pstart[-1] + padded_g[-1]) // TILE).astype(jnp.int32)[None]
    tile_expert = (
        jnp.searchsorted(pstart, jnp.arange(NT, dtype=jnp.int32) * TILE,
                         side="right").astype(jnp.int32) - 1)
    slot_ar = jnp.arange(NTPAD, dtype=jnp.int32)
    dest = jnp.where(rows < T, rows, TRASH + slot_ar // BPW)
    prob_pad = jnp.zeros((NTPAD, 1), jnp.float32).at[dest_slot, 0].set(prob_vec)

    x_pad = _dispatch(x2d, dest_slot)
    w1r = W1
    b1r = b1.reshape(E, 1, F)
    w2r = W2
    b2r = b2.reshape(E, 1, D)
    y_pad = _ffn(x_pad, w1r, b1r, w2r, b2r, prob_pad, tile_expert, nactive)
    y_all = _combine(y_pad, dest)

    hidden = y_all[:T].reshape(b, s, d)
    expert_index = eidx.reshape(b, s)
    return hidden, router_logits, expert_index


# speedup vs baseline: 1.0949x; 1.0949x over previous
import functools

import jax
import jax.numpy as jnp
from jax import lax
from jax.experimental import pallas as pl
from jax.experimental.pallas import tpu as pltpu
from jax.experimental.pallas import tpu_sc as plsc

T = 2048
D = 1024
E = 8
F = 4096
EPAD = 128
TILE = 256
NT = 15
NTPAD = NT * TILE
NW = 32
BPW = NTPAD // NW
TRASH = T
NJ = 2


def _router_body(x_ref, wr_ref, logits_ref, eidx_ref, prob_ref):
    x = x_ref[...].astype(jnp.bfloat16)
    wr = wr_ref[...].astype(jnp.bfloat16)
    logits = jnp.dot(x, wr, preferred_element_type=jnp.float32)
    lane = lax.broadcasted_iota(jnp.int32, (T, EPAD), 1)
    masked = jnp.where(lane < E, logits, -jnp.inf)
    m = jnp.max(masked, axis=-1, keepdims=True)
    ex = jnp.exp(masked - m)
    s = jnp.sum(ex, axis=-1, keepdims=True)
    probs = ex / s
    pmax = jnp.max(probs, axis=-1, keepdims=True)
    eidx = jnp.min(jnp.where(probs == pmax, lane, EPAD), axis=-1, keepdims=True)
    logits_ref[...] = logits
    eidx_ref[...] = jnp.broadcast_to(eidx, (T, EPAD))
    prob_ref[...] = jnp.broadcast_to(pmax, (T, EPAD))


def _router(x2d, wr_pad):
    return pl.pallas_call(
        _router_body,
        out_shape=(
            jax.ShapeDtypeStruct((T, EPAD), jnp.float32),
            jax.ShapeDtypeStruct((T, EPAD), jnp.int32),
            jax.ShapeDtypeStruct((T, EPAD), jnp.float32),
        ),
    )(x2d, wr_pad)


TPW = T // NW


def _sc_dispatch_body(x_hbm, idx_hbm, out_hbm, idx_v, rows_v, sem):
    wid = lax.axis_index("s") * 2 + lax.axis_index("c")
    base = wid * TPW
    pltpu.sync_copy(idx_hbm.at[pl.ds(base, TPW)], idx_v)
    pltpu.sync_copy(x_hbm.at[pl.ds(base, TPW)], rows_v)
    pltpu.async_copy(rows_v, out_hbm.at[idx_v], sem).wait()


@functools.cache
def _dispatch_kernel():
    return pl.kernel(
        _sc_dispatch_body,
        out_type=jax.ShapeDtypeStruct((NTPAD, D), jnp.float32),
        mesh=plsc.VectorSubcoreMesh(core_axis_name="c", subcore_axis_name="s"),
        scratch_types=[
            pltpu.VMEM((TPW,), jnp.int32),
            pltpu.VMEM((TPW, D), jnp.float32),
            pltpu.SemaphoreType.DMA,
        ],
    )


def _dispatch(x2d, dest_slot):
    return _dispatch_kernel()(x2d, dest_slot)


def _sc_scatter_body(ypad_hbm, idx_hbm, out_hbm, idx_v, rows_v, sem):
    wid = lax.axis_index("s") * 2 + lax.axis_index("c")
    base = wid * BPW
    pltpu.sync_copy(idx_hbm.at[pl.ds(base, BPW)], idx_v)
    pltpu.sync_copy(ypad_hbm.at[pl.ds(base, BPW)], rows_v)
    pltpu.async_copy(rows_v, out_hbm.at[idx_v], sem).wait()


@functools.cache
def _combine_kernel():
    return pl.kernel(
        _sc_scatter_body,
        out_type=jax.ShapeDtypeStruct((T + NW, D), jnp.float32),
        mesh=plsc.VectorSubcoreMesh(core_axis_name="c", subcore_axis_name="s"),
        scratch_types=[
            pltpu.VMEM((BPW,), jnp.int32),
            pltpu.VMEM((BPW, D), jnp.float32),
            pltpu.SemaphoreType.DMA,
        ],
    )


def _combine(y_pad, dest):
    return _combine_kernel()(y_pad, dest)


def _ffn_body(te_ref, na_ref, x_ref, w1_ref, b1_ref, w2_ref, b2_ref, p_ref,
              y_ref, acc_ref):
    j = pl.program_id(0)
    t = pl.program_id(1)

    @pl.when(t < na_ref[0])
    def _():
        xb = x_ref[...].astype(jnp.bfloat16)
        w1 = w1_ref[0].astype(jnp.bfloat16)
        h = jnp.dot(xb, w1, preferred_element_type=jnp.float32) + b1_ref[0]
        h = jnp.maximum(h, 0.0)
        w2 = w2_ref[0].astype(jnp.bfloat16)
        o = jnp.dot(h.astype(jnp.bfloat16), w2, preferred_element_type=jnp.float32)
        sl = pl.ds(t * TILE, TILE)

        @pl.when(j == 0)
        def _():
            acc_ref[sl, :] = o

        @pl.when(jnp.logical_and(j > 0, j < NJ - 1))
        def _():
            acc_ref[sl, :] += o

        @pl.when(j == NJ - 1)
        def _():
            y_ref[...] = (acc_ref[sl, :] + o + b2_ref[0]) * p_ref[...]


def _ffn(x_pad, w1r, b1r, w2r, b2r, prob_pad, tile_expert, nactive):
    fh = F // NJ
    grid_spec = pltpu.PrefetchScalarGridSpec(
        num_scalar_prefetch=2,
        grid=(NJ, NT),
        in_specs=[
            pl.BlockSpec((TILE, D), lambda j, t, te, na: (t, 0)),
            pl.BlockSpec((1, D, fh), lambda j, t, te, na: (te[t], 0, j)),
            pl.BlockSpec((1, 1, fh), lambda j, t, te, na: (te[t], 0, j)),
            pl.BlockSpec((1, fh, D), lambda j, t, te, na: (te[t], j, 0)),
            pl.BlockSpec((1, 1, D), lambda j, t, te, na: (te[t], 0, 0)),
            pl.BlockSpec((TILE, 1), lambda j, t, te, na: (t, 0)),
        ],
        out_specs=pl.BlockSpec(
            (TILE, D), lambda j, t, te, na: (jnp.where(j < NJ - 1, NT, t), 0)),
        scratch_shapes=[pltpu.VMEM((NTPAD, D), jnp.float32)],
    )
    out = pl.pallas_call(
        _ffn_body,
        grid_spec=grid_spec,
        out_shape=jax.ShapeDtypeStruct((NTPAD + TILE, D), jnp.float32),
    )(tile_expert, nactive, x_pad, w1r, b1r, w2r, b2r, prob_pad)
    return out[:NTPAD]


def kernel(hidden_states, Wr, W1, b1, W2, b2):
    b, s, d = hidden_states.shape
    x2d = hidden_states.reshape(T, D)

    wr_pad = jnp.zeros((D, EPAD), jnp.float32).at[:, :E].set(Wr)
    logits_pad, eidx_b, prob_b = _router(x2d, wr_pad)
    eidx = eidx_b[:, 0]
    prob_vec = prob_b[:, 0]
    router_logits = logits_pad[:, :E].reshape(b, s, E)

    onehot = (eidx[:, None] == jnp.arange(E)[None, :]).astype(jnp.int32)
    csum = jnp.cumsum(onehot, axis=0)
    rank = jnp.take_along_axis(csum, eidx[:, None], axis=1)[:, 0] - 1
    g = csum[-1]
    padded_g = ((g + (TILE - 1)) // TILE) * TILE
    pstart = jnp.concatenate(
        [jnp.zeros((1,), jnp.int32), jnp.cumsum(padded_g)[:-1]])
    dest_slot = pstart[eidx] + rank
    rows = jnp.full((NTPAD,), T, jnp.int32).at[dest_slot].set(
        jnp.arange(T, dtype=jnp.int32))
    nactive = ((pstart[-1] + padded_g[-1]) // TILE).astype(jnp.int32)[None]
    tile_expert = (
        jnp.searchsorted(pstart, jnp.arange(NT, dtype=jnp.int32) * TILE,
                         side="right").astype(jnp.int32) - 1)
    slot_ar = jnp.arange(NTPAD, dtype=jnp.int32)
    dest = jnp.where(rows < T, rows, TRASH + slot_ar // BPW)
    prob_pad = jnp.zeros((NTPAD, 1), jnp.float32).at[dest_slot, 0].set(prob_vec)

    x_pad = _dispatch(x2d, dest_slot)
    w1r = W1
    b1r = b1.reshape(E, 1, F)
    w2r = W2
    b2r = b2.reshape(E, 1, D)
    y_pad = _ffn(x_pad, w1r, b1r, w2r, b2r, prob_pad, tile_expert, nactive)
    y_all = _combine(y_pad, dest)

    hidden = y_all[:T].reshape(b, s, d)
    expert_index = eidx.reshape(b, s)
    return hidden, router_logits, expert_index
